# baseline (speedup 1.0000x reference)
import functools

import jax
import jax.numpy as jnp
from jax import lax
from jax.experimental import pallas as pl
from jax.experimental.pallas import tpu as pltpu
from jax.experimental.pallas import tpu_sc as plsc

N = 50000
E = 800000
G = 256
NP = 50176
EP = 800256
R = 512
C = 128
B = 64
NBT = NP // B
NW = 32
SUB_ITERS = (NBT + NW - 1) // NW


def _mm_scale_body(x_ref, w_ref, deg_ref, o_ref):
    d = lax.rsqrt(deg_ref[:, :1])
    o_ref[...] = jnp.dot(x_ref[...], w_ref[...],
                         preferred_element_type=jnp.float32) * d


def _matmul_scale(xp, w, degb, dout):
    din = xp.shape[1]
    return pl.pallas_call(
        _mm_scale_body,
        grid=(NP // R,),
        in_specs=[pl.BlockSpec((R, din), lambda i: (i, 0)),
                  pl.BlockSpec((din, dout), lambda i: (0, 0)),
                  pl.BlockSpec((R, 128), lambda i: (i, 0))],
        out_specs=pl.BlockSpec((R, dout), lambda i: (i, 0)),
        out_shape=jax.ShapeDtypeStruct((NP, dout), jnp.float32),
    )(xp, w, degb)


def _post_mm_body(valid, agg_ref, y_ref, deg_ref, b_ref, g_ref, bt_ref,
                  w_ref, o_ref):
    d = lax.rsqrt(deg_ref[:, :1])
    v = jnp.maximum(d * (agg_ref[...] + y_ref[...]) + b_ref[...], 0.0)
    dp = v.shape[1]
    inv = 1.0 / valid
    mu = jnp.sum(v, axis=1, keepdims=True) * inv
    mask = (lax.broadcasted_iota(jnp.int32, (1, dp), 1) < valid
            ).astype(jnp.float32)
    c = (v - mu) * mask
    var = jnp.sum(c * c, axis=1, keepdims=True) * inv
    h = c * lax.rsqrt(var + 1e-5) * g_ref[...] + bt_ref[...]
    o_ref[...] = jnp.dot(h, w_ref[...],
                         preferred_element_type=jnp.float32) * d


def _post_mm(valid, agg, y, degb, bp, gp, btp, w, dout):
    dp = y.shape[1]
    return pl.pallas_call(
        functools.partial(_post_mm_body, float(valid)),
        grid=(NP // R,),
        in_specs=[pl.BlockSpec((R, dp), lambda i: (i, 0)),
                  pl.BlockSpec((R, dp), lambda i: (i, 0)),
                  pl.BlockSpec((R, 128), lambda i: (i, 0)),
                  pl.BlockSpec((1, dp), lambda i: (0, 0)),
                  pl.BlockSpec((1, dp), lambda i: (0, 0)),
                  pl.BlockSpec((1, dp), lambda i: (0, 0)),
                  pl.BlockSpec((dp, dout), lambda i: (0, 0))],
        out_specs=pl.BlockSpec((R, dout), lambda i: (i, 0)),
        out_shape=jax.ShapeDtypeStruct((NP, dout), jnp.float32),
    )(agg, y, degb, bp, gp, btp, w)


def _post_pool_body(agg_ref, y_ref, deg_ref, b_ref, g_ref, bt_ref,
                    batch_ref, ps_ref, cnt_ref):
    i = pl.program_id(0)
    d = lax.rsqrt(deg_ref[:, :1])
    v = jnp.maximum(d * (agg_ref[...] + y_ref[...]) + b_ref[...], 0.0)
    mu = jnp.mean(v, axis=1, keepdims=True)
    c = v - mu
    var = jnp.mean(c * c, axis=1, keepdims=True)
    h = c * lax.rsqrt(var + 1e-5) * g_ref[...] + bt_ref[...]
    onehot = (batch_ref[:, :1] == lax.broadcasted_iota(jnp.int32, (1, G), 1)
              ).astype(jnp.float32)
    dn = (((0,), (0,)), ((), ()))
    ps = lax.dot_general(onehot, h, dn, preferred_element_type=jnp.float32)
    cn = lax.dot_general(onehot, jnp.ones((R, 128), jnp.float32), dn,
                         preferred_element_type=jnp.float32)

    @pl.when(i == 0)
    def _():
        ps_ref[...] = ps
        cnt_ref[...] = cn

    @pl.when(i != 0)
    def _():
        ps_ref[...] += ps
        cnt_ref[...] += cn


def _post_pool(agg, y, degb, bp, gp, btp, batchb):
    return pl.pallas_call(
        _post_pool_body,
        grid=(NP // R,),
        in_specs=[pl.BlockSpec((R, 128), lambda i: (i, 0)),
                  pl.BlockSpec((R, 128), lambda i: (i, 0)),
                  pl.BlockSpec((R, 128), lambda i: (i, 0)),
                  pl.BlockSpec((1, 128), lambda i: (0, 0)),
                  pl.BlockSpec((1, 128), lambda i: (0, 0)),
                  pl.BlockSpec((1, 128), lambda i: (0, 0)),
                  pl.BlockSpec((R, 128), lambda i: (i, 0))],
        out_specs=[pl.BlockSpec((G, 128), lambda i: (0, 0)),
                   pl.BlockSpec((G, 128), lambda i: (0, 0))],
        out_shape=[jax.ShapeDtypeStruct((G, 128), jnp.float32),
                   jax.ShapeDtypeStruct((G, 128), jnp.float32)],
    )(agg, y, degb, bp, gp, btp, batchb)


def _fc_body(ps_ref, cnt_ref, w1_ref, b1_ref, w2_ref, b2_ref, o_ref):
    pooled = ps_ref[...] / jnp.maximum(cnt_ref[...], 1.0)
    z1 = jnp.maximum(
        jnp.dot(pooled, w1_ref[...], preferred_element_type=jnp.float32)
        + b1_ref[...], 0.0)
    o_ref[...] = jnp.dot(z1, w2_ref[...],
                         preferred_element_type=jnp.float32) + b2_ref[...]


def _fc(ps, cnt, w1, b1, w2, b2):
    return pl.pallas_call(
        _fc_body,
        out_shape=jax.ShapeDtypeStruct((G, 128), jnp.float32),
    )(ps, cnt, w1, b1, w2, b2)


def _make_agg(D):
    mesh = plsc.VectorSubcoreMesh(core_axis_name="c", subcore_axis_name="s")

    @functools.partial(
        pl.kernel,
        out_type=jax.ShapeDtypeStruct((NP, D), jnp.float32),
        mesh=mesh,
        compiler_params=pltpu.CompilerParams(use_tc_tiling_on_sc=False,
                                             needs_layout_passes=False),
        scratch_types=[
            pltpu.VMEM((C,), jnp.int32),
            pltpu.VMEM((C,), jnp.int32),
            pltpu.VMEM((C,), jnp.int32),
            pltpu.VMEM((C, D), jnp.float32),
            pltpu.VMEM((B + 8, D), jnp.float32),
            pltpu.VMEM((16,), jnp.int32),
            pltpu.VMEM((16,), jnp.int32),
            pltpu.SemaphoreType.DMA,
        ],
    )
    def agg_kernel(y_hbm, src_hbm, dst_hbm, ro_hbm, out_hbm,
                   sbuf, dbuf, dloc, gbuf, acc, ro0, ro1, sem):
        wid = lax.axis_index("s") * 2 + lax.axis_index("c")
        iota16 = lax.broadcasted_iota(jnp.int32, (16,), 0)
        zero16 = jnp.zeros((16,), jnp.float32)

        def subblock(it, carry):
            nb = it * NW + wid

            @pl.when(nb < NBT)
            def _():
                base = nb * B

                def zrow(r, c2):
                    for j in range(D // 16):
                        acc[r, pl.ds(j * 16, 16)] = zero16
                    return c2

                lax.fori_loop(0, B, zrow, 0)
                pltpu.sync_copy(ro_hbm.at[pl.ds(base, 16)], ro0)
                pltpu.sync_copy(ro_hbm.at[pl.ds(base + B, 16)], ro1)
                s = ro0[...][0]
                t = ro1[...][0]
                a0 = (s // 8) * 8
                nch = (t - a0 + (C - 1)) // C

                def chunk(ci, c2):
                    a = a0 + ci * C
                    pltpu.sync_copy(src_hbm.at[pl.ds(a, C)], sbuf)
                    pltpu.sync_copy(dst_hbm.at[pl.ds(a, C)], dbuf)
                    for k in range(C // 16):
                        d16 = dbuf[pl.ds(k * 16, 16)]
                        e16 = (a + k * 16) + iota16
                        ok = (e16 >= s) & (e16 < t)
                        dloc[pl.ds(k * 16, 16)] = jnp.where(ok, d16 - base, B)
                    pltpu.async_copy(y_hbm.at[sbuf], gbuf, sem).wait()
                    for k in range(C // 16):
                        dl16 = dloc[pl.ds(k * 16, 16)]
                        ri = k * 16 + iota16

                        def jloop(j0, c3):
                            for jj in range(16):
                                js = jnp.full((16,), 1, jnp.int32) * (
                                    j0 * 16 + jj)
                                xcol = plsc.load_gather(gbuf, [ri, js])
                                plsc.addupdate_scatter(acc, [dl16, js], xcol)
                            return c3

                        lax.fori_loop(0, D // 16, jloop, 0)
                    return c2

                lax.fori_loop(0, nch, chunk, 0)
                pltpu.sync_copy(acc.at[pl.ds(0, B)],
                                out_hbm.at[pl.ds(base, B)])

            return carry

        lax.fori_loop(0, SUB_ITERS, subblock, 0)

    return agg_kernel


_agg256 = _make_agg(256)
_agg384 = _make_agg(384)
_agg128 = _make_agg(128)


def kernel(x, edge_index, batch, W1, b1, g1, bt1, W2, b2, g2, bt2,
           W3, b3, g3, bt3, fW1, fb1, fW2, fb2):
    f32 = jnp.float32
    src = edge_index[0]
    dst = edge_index[1]
    dst_s, src_s = lax.sort_key_val(dst, src)
    ro = jnp.searchsorted(
        dst_s, jnp.arange(NP + 16, dtype=jnp.int32), side='left'
    ).astype(jnp.int32)
    src_p = jnp.pad(src_s, (0, EP - E))
    dst_p = jnp.pad(dst_s, (0, EP - E), constant_values=N)
    deg = (ro[1:NP + 1] - ro[:NP]).astype(f32) + 1.0
    degb = jnp.broadcast_to(deg[:, None], (NP, 128))
    batch_p = jnp.pad(batch, (0, NP - N), constant_values=G)
    batchb = jnp.broadcast_to(batch_p[:, None], (NP, 128))

    xp = jnp.pad(x, ((0, NP - N), (0, 128 - x.shape[1])))
    W1p = jnp.pad(W1, ((0, 128 - W1.shape[0]), (0, 256 - W1.shape[1])))
    b1p = jnp.pad(b1, (0, 256 - b1.shape[0]))[None, :]
    g1p = jnp.pad(g1, (0, 256 - g1.shape[0]))[None, :]
    bt1p = jnp.pad(bt1, (0, 256 - bt1.shape[0]))[None, :]
    W2p = jnp.pad(W2, ((0, 256 - W2.shape[0]), (0, 384 - W2.shape[1])))
    b2p = jnp.pad(b2, (0, 384 - b2.shape[0]))[None, :]
    g2p = jnp.pad(g2, (0, 384 - g2.shape[0]))[None, :]
    bt2p = jnp.pad(bt2, (0, 384 - bt2.shape[0]))[None, :]
    W3p = jnp.pad(W3, ((0, 384 - W3.shape[0]), (0, 0)))

    y1 = _matmul_scale(xp, W1p, degb, 256)
    agg1 = _agg256(y1, src_p, dst_p, ro)
    y2 = _post_mm(156, agg1, y1, degb, b1p, g1p, bt1p, W2p, 384)
    agg2 = _agg384(y2, src_p, dst_p, ro)
    y3 = _post_mm(312, agg2, y2, degb, b2p, g2p, bt2p, W3p, 128)
    agg3 = _agg128(y3, src_p, dst_p, ro)
    ps, cnt = _post_pool(agg3, y3, degb, b3[None, :], g3[None, :],
                         bt3[None, :], batchb)
    return _fc(ps, cnt, fW1, fb1[None, :], fW2, fb2[None, :])


# speedup vs baseline: 3.5395x; 3.5395x over previous
import functools

import jax
import jax.numpy as jnp
from jax import lax
from jax.experimental import pallas as pl
from jax.experimental.pallas import tpu as pltpu
from jax.experimental.pallas import tpu_sc as plsc

N = 50000
E = 800000
G = 256
NP = 50176
EP = 800256
R = 512
C = 128
B = 64
NBT = NP // B
NW = 32
SUB_ITERS = (NBT + NW - 1) // NW


def _mm_scale_body(x_ref, w_ref, deg_ref, o_ref):
    d = lax.rsqrt(deg_ref[:, :1])
    o_ref[...] = jnp.dot(x_ref[...], w_ref[...],
                         preferred_element_type=jnp.float32) * d


def _matmul_scale(xp, w, degb, dout):
    din = xp.shape[1]
    return pl.pallas_call(
        _mm_scale_body,
        grid=(NP // R,),
        in_specs=[pl.BlockSpec((R, din), lambda i: (i, 0)),
                  pl.BlockSpec((din, dout), lambda i: (0, 0)),
                  pl.BlockSpec((R, 128), lambda i: (i, 0))],
        out_specs=pl.BlockSpec((R, dout), lambda i: (i, 0)),
        out_shape=jax.ShapeDtypeStruct((NP, dout), jnp.float32),
    )(xp, w, degb)


def _post_mm_body(valid, agg_ref, y_ref, deg_ref, b_ref, g_ref, bt_ref,
                  w_ref, o_ref):
    d = lax.rsqrt(deg_ref[:, :1])
    v = jnp.maximum(d * (agg_ref[...] + y_ref[...]) + b_ref[...], 0.0)
    dp = v.shape[1]
    inv = 1.0 / valid
    mu = jnp.sum(v, axis=1, keepdims=True) * inv
    mask = (lax.broadcasted_iota(jnp.int32, (1, dp), 1) < valid
            ).astype(jnp.float32)
    c = (v - mu) * mask
    var = jnp.sum(c * c, axis=1, keepdims=True) * inv
    h = c * lax.rsqrt(var + 1e-5) * g_ref[...] + bt_ref[...]
    o_ref[...] = jnp.dot(h, w_ref[...],
                         preferred_element_type=jnp.float32) * d


def _post_mm(valid, agg, y, degb, bp, gp, btp, w, dout):
    dp = y.shape[1]
    return pl.pallas_call(
        functools.partial(_post_mm_body, float(valid)),
        grid=(NP // R,),
        in_specs=[pl.BlockSpec((R, dp), lambda i: (i, 0)),
                  pl.BlockSpec((R, dp), lambda i: (i, 0)),
                  pl.BlockSpec((R, 128), lambda i: (i, 0)),
                  pl.BlockSpec((1, dp), lambda i: (0, 0)),
                  pl.BlockSpec((1, dp), lambda i: (0, 0)),
                  pl.BlockSpec((1, dp), lambda i: (0, 0)),
                  pl.BlockSpec((dp, dout), lambda i: (0, 0))],
        out_specs=pl.BlockSpec((R, dout), lambda i: (i, 0)),
        out_shape=jax.ShapeDtypeStruct((NP, dout), jnp.float32),
    )(agg, y, degb, bp, gp, btp, w)


def _post_pool_body(agg_ref, y_ref, deg_ref, b_ref, g_ref, bt_ref,
                    batch_ref, ps_ref, cnt_ref):
    i = pl.program_id(0)
    d = lax.rsqrt(deg_ref[:, :1])
    v = jnp.maximum(d * (agg_ref[...] + y_ref[...]) + b_ref[...], 0.0)
    mu = jnp.mean(v, axis=1, keepdims=True)
    c = v - mu
    var = jnp.mean(c * c, axis=1, keepdims=True)
    h = c * lax.rsqrt(var + 1e-5) * g_ref[...] + bt_ref[...]
    onehot = (batch_ref[:, :1] == lax.broadcasted_iota(jnp.int32, (1, G), 1)
              ).astype(jnp.float32)
    dn = (((0,), (0,)), ((), ()))
    ps = lax.dot_general(onehot, h, dn, preferred_element_type=jnp.float32)
    cn = lax.dot_general(onehot, jnp.ones((R, 128), jnp.float32), dn,
                         preferred_element_type=jnp.float32)

    @pl.when(i == 0)
    def _():
        ps_ref[...] = ps
        cnt_ref[...] = cn

    @pl.when(i != 0)
    def _():
        ps_ref[...] += ps
        cnt_ref[...] += cn


def _post_pool(agg, y, degb, bp, gp, btp, batchb):
    return pl.pallas_call(
        _post_pool_body,
        grid=(NP // R,),
        in_specs=[pl.BlockSpec((R, 128), lambda i: (i, 0)),
                  pl.BlockSpec((R, 128), lambda i: (i, 0)),
                  pl.BlockSpec((R, 128), lambda i: (i, 0)),
                  pl.BlockSpec((1, 128), lambda i: (0, 0)),
                  pl.BlockSpec((1, 128), lambda i: (0, 0)),
                  pl.BlockSpec((1, 128), lambda i: (0, 0)),
                  pl.BlockSpec((R, 128), lambda i: (i, 0))],
        out_specs=[pl.BlockSpec((G, 128), lambda i: (0, 0)),
                   pl.BlockSpec((G, 128), lambda i: (0, 0))],
        out_shape=[jax.ShapeDtypeStruct((G, 128), jnp.float32),
                   jax.ShapeDtypeStruct((G, 128), jnp.float32)],
    )(agg, y, degb, bp, gp, btp, batchb)


def _fc_body(ps_ref, cnt_ref, w1_ref, b1_ref, w2_ref, b2_ref, o_ref):
    pooled = ps_ref[...] / jnp.maximum(cnt_ref[...], 1.0)
    z1 = jnp.maximum(
        jnp.dot(pooled, w1_ref[...], preferred_element_type=jnp.float32)
        + b1_ref[...], 0.0)
    o_ref[...] = jnp.dot(z1, w2_ref[...],
                         preferred_element_type=jnp.float32) + b2_ref[...]


def _fc(ps, cnt, w1, b1, w2, b2):
    return pl.pallas_call(
        _fc_body,
        out_shape=jax.ShapeDtypeStruct((G, 128), jnp.float32),
    )(ps, cnt, w1, b1, w2, b2)


def _make_agg(D):
    mesh = plsc.VectorSubcoreMesh(core_axis_name="c", subcore_axis_name="s")

    @functools.partial(
        pl.kernel,
        out_type=jax.ShapeDtypeStruct((NP, D), jnp.float32),
        mesh=mesh,
        compiler_params=pltpu.CompilerParams(use_tc_tiling_on_sc=False,
                                             needs_layout_passes=False),
        scratch_types=[
            pltpu.VMEM((C,), jnp.int32),
            pltpu.VMEM((C + 16,), jnp.int32),
            pltpu.VMEM((C, D), jnp.float32),
            pltpu.VMEM((B, D), jnp.float32),
            pltpu.VMEM((B + 16,), jnp.int32),
            pltpu.SemaphoreType.DMA,
        ],
    )
    def agg_kernel(y_hbm, src_hbm, dst_hbm, ro_hbm, out_hbm,
                   sbuf, dbuf, gbuf, acc, robuf, sem):
        wid = lax.axis_index("s") * 2 + lax.axis_index("c")
        zero16 = jnp.zeros((16,), jnp.float32)

        def subblock(it, carry):
            nb = it * NW + wid

            @pl.when(nb < NBT)
            def _():
                base = nb * B

                def zrow(r, c2):
                    for j in range(D // 16):
                        acc[r, pl.ds(j * 16, 16)] = zero16
                    return c2

                lax.fori_loop(0, B, zrow, 0)
                pltpu.sync_copy(ro_hbm.at[pl.ds(base, B + 16)], robuf)
                s = robuf[pl.ds(0, 16)][0]
                t = robuf[pl.ds(B, 16)][0]
                a0 = (s // 8) * 8
                nch = (t - a0 + (C - 1)) // C

                def chunk(ci, c2):
                    a = a0 + ci * C
                    pltpu.sync_copy(src_hbm.at[pl.ds(a, C)], sbuf)
                    pltpu.sync_copy(dst_hbm.at[pl.ds(a, C)],
                                    dbuf.at[pl.ds(0, C)])
                    pltpu.async_copy(y_hbm.at[sbuf], gbuf, sem).wait()
                    elo = jnp.maximum(s - a, 0)
                    ehi = jnp.minimum(t - a, C)

                    def edge(e, c3):
                        dv = dbuf[pl.ds(e, 16)][0] - base
                        for j in range(D // 16):
                            sl = pl.ds(j * 16, 16)
                            plsc.addupdate(acc.at[dv, sl], gbuf[e, sl])
                        return c3

                    lax.fori_loop(elo, ehi, edge, 0)
                    return c2

                lax.fori_loop(0, nch, chunk, 0)
                pltpu.sync_copy(acc.at[pl.ds(0, B)],
                                out_hbm.at[pl.ds(base, B)])

            return carry

        lax.fori_loop(0, SUB_ITERS, subblock, 0)

    return agg_kernel


_agg256 = _make_agg(256)
_agg384 = _make_agg(384)
_agg128 = _make_agg(128)


def kernel(x, edge_index, batch, W1, b1, g1, bt1, W2, b2, g2, bt2,
           W3, b3, g3, bt3, fW1, fb1, fW2, fb2):
    f32 = jnp.float32
    src = edge_index[0]
    dst = edge_index[1]
    dst_s, src_s = lax.sort_key_val(dst, src)
    ro = jnp.searchsorted(
        dst_s, jnp.arange(NP + 16, dtype=jnp.int32), side='left'
    ).astype(jnp.int32)
    src_p = jnp.pad(src_s, (0, EP - E))
    dst_p = jnp.pad(dst_s, (0, EP - E), constant_values=N)
    deg = (ro[1:NP + 1] - ro[:NP]).astype(f32) + 1.0
    degb = jnp.broadcast_to(deg[:, None], (NP, 128))
    batch_p = jnp.pad(batch, (0, NP - N), constant_values=G)
    batchb = jnp.broadcast_to(batch_p[:, None], (NP, 128))

    xp = jnp.pad(x, ((0, NP - N), (0, 128 - x.shape[1])))
    W1p = jnp.pad(W1, ((0, 128 - W1.shape[0]), (0, 256 - W1.shape[1])))
    b1p = jnp.pad(b1, (0, 256 - b1.shape[0]))[None, :]
    g1p = jnp.pad(g1, (0, 256 - g1.shape[0]))[None, :]
    bt1p = jnp.pad(bt1, (0, 256 - bt1.shape[0]))[None, :]
    W2p = jnp.pad(W2, ((0, 256 - W2.shape[0]), (0, 384 - W2.shape[1])))
    b2p = jnp.pad(b2, (0, 384 - b2.shape[0]))[None, :]
    g2p = jnp.pad(g2, (0, 384 - g2.shape[0]))[None, :]
    bt2p = jnp.pad(bt2, (0, 384 - bt2.shape[0]))[None, :]
    W3p = jnp.pad(W3, ((0, 384 - W3.shape[0]), (0, 0)))

    y1 = _matmul_scale(xp, W1p, degb, 256)
    agg1 = _agg256(y1, src_p, dst_p, ro)
    y2 = _post_mm(156, agg1, y1, degb, b1p, g1p, bt1p, W2p, 384)
    agg2 = _agg384(y2, src_p, dst_p, ro)
    y3 = _post_mm(312, agg2, y2, degb, b2p, g2p, bt2p, W3p, 128)
    agg3 = _agg128(y3, src_p, dst_p, ro)
    ps, cnt = _post_pool(agg3, y3, degb, b3[None, :], g3[None, :],
                         bt3[None, :], batchb)
    return _fc(ps, cnt, fW1, fb1[None, :], fW2, fb2[None, :])


# speedup vs baseline: 4.0034x; 1.1311x over previous
import functools

import jax
import jax.numpy as jnp
from jax import lax
from jax.experimental import pallas as pl
from jax.experimental.pallas import tpu as pltpu
from jax.experimental.pallas import tpu_sc as plsc

N = 50000
E = 800000
G = 256
NP = 50176
EP = 800256
R = 512
C = 128
B = 64
NBT = NP // B
NW = 32
SUB_ITERS = (NBT + NW - 1) // NW


def _mm_scale_body(x_ref, w_ref, deg_ref, o_ref):
    d = lax.rsqrt(deg_ref[:, :1])
    o_ref[...] = jnp.dot(x_ref[...], w_ref[...],
                         preferred_element_type=jnp.float32) * d


def _matmul_scale(xp, w, degb, dout):
    din = xp.shape[1]
    return pl.pallas_call(
        _mm_scale_body,
        grid=(NP // R,),
        in_specs=[pl.BlockSpec((R, din), lambda i: (i, 0)),
                  pl.BlockSpec((din, dout), lambda i: (0, 0)),
                  pl.BlockSpec((R, 128), lambda i: (i, 0))],
        out_specs=pl.BlockSpec((R, dout), lambda i: (i, 0)),
        out_shape=jax.ShapeDtypeStruct((NP, dout), jnp.float32),
    )(xp, w, degb)


def _post_mm_body(valid, agg_ref, y_ref, deg_ref, b_ref, g_ref, bt_ref,
                  w_ref, o_ref):
    d = lax.rsqrt(deg_ref[:, :1])
    v = jnp.maximum(d * (agg_ref[...] + y_ref[...]) + b_ref[...], 0.0)
    dp = v.shape[1]
    inv = 1.0 / valid
    mu = jnp.sum(v, axis=1, keepdims=True) * inv
    mask = (lax.broadcasted_iota(jnp.int32, (1, dp), 1) < valid
            ).astype(jnp.float32)
    c = (v - mu) * mask
    var = jnp.sum(c * c, axis=1, keepdims=True) * inv
    h = c * lax.rsqrt(var + 1e-5) * g_ref[...] + bt_ref[...]
    o_ref[...] = jnp.dot(h, w_ref[...],
                         preferred_element_type=jnp.float32) * d


def _post_mm(valid, agg, y, degb, bp, gp, btp, w, dout):
    dp = y.shape[1]
    return pl.pallas_call(
        functools.partial(_post_mm_body, float(valid)),
        grid=(NP // R,),
        in_specs=[pl.BlockSpec((R, dp), lambda i: (i, 0)),
                  pl.BlockSpec((R, dp), lambda i: (i, 0)),
                  pl.BlockSpec((R, 128), lambda i: (i, 0)),
                  pl.BlockSpec((1, dp), lambda i: (0, 0)),
                  pl.BlockSpec((1, dp), lambda i: (0, 0)),
                  pl.BlockSpec((1, dp), lambda i: (0, 0)),
                  pl.BlockSpec((dp, dout), lambda i: (0, 0))],
        out_specs=pl.BlockSpec((R, dout), lambda i: (i, 0)),
        out_shape=jax.ShapeDtypeStruct((NP, dout), jnp.float32),
    )(agg, y, degb, bp, gp, btp, w)


def _post_pool_body(agg_ref, y_ref, deg_ref, b_ref, g_ref, bt_ref,
                    batch_ref, ps_ref, cnt_ref):
    i = pl.program_id(0)
    d = lax.rsqrt(deg_ref[:, :1])
    v = jnp.maximum(d * (agg_ref[...] + y_ref[...]) + b_ref[...], 0.0)
    mu = jnp.mean(v, axis=1, keepdims=True)
    c = v - mu
    var = jnp.mean(c * c, axis=1, keepdims=True)
    h = c * lax.rsqrt(var + 1e-5) * g_ref[...] + bt_ref[...]
    onehot = (batch_ref[:, :1] == lax.broadcasted_iota(jnp.int32, (1, G), 1)
              ).astype(jnp.float32)
    dn = (((0,), (0,)), ((), ()))
    ps = lax.dot_general(onehot, h, dn, preferred_element_type=jnp.float32)
    cn = lax.dot_general(onehot, jnp.ones((R, 128), jnp.float32), dn,
                         preferred_element_type=jnp.float32)

    @pl.when(i == 0)
    def _():
        ps_ref[...] = ps
        cnt_ref[...] = cn

    @pl.when(i != 0)
    def _():
        ps_ref[...] += ps
        cnt_ref[...] += cn


def _post_pool(agg, y, degb, bp, gp, btp, batchb):
    return pl.pallas_call(
        _post_pool_body,
        grid=(NP // R,),
        in_specs=[pl.BlockSpec((R, 128), lambda i: (i, 0)),
                  pl.BlockSpec((R, 128), lambda i: (i, 0)),
                  pl.BlockSpec((R, 128), lambda i: (i, 0)),
                  pl.BlockSpec((1, 128), lambda i: (0, 0)),
                  pl.BlockSpec((1, 128), lambda i: (0, 0)),
                  pl.BlockSpec((1, 128), lambda i: (0, 0)),
                  pl.BlockSpec((R, 128), lambda i: (i, 0))],
        out_specs=[pl.BlockSpec((G, 128), lambda i: (0, 0)),
                   pl.BlockSpec((G, 128), lambda i: (0, 0))],
        out_shape=[jax.ShapeDtypeStruct((G, 128), jnp.float32),
                   jax.ShapeDtypeStruct((G, 128), jnp.float32)],
    )(agg, y, degb, bp, gp, btp, batchb)


def _fc_body(ps_ref, cnt_ref, w1_ref, b1_ref, w2_ref, b2_ref, o_ref):
    pooled = ps_ref[...] / jnp.maximum(cnt_ref[...], 1.0)
    z1 = jnp.maximum(
        jnp.dot(pooled, w1_ref[...], preferred_element_type=jnp.float32)
        + b1_ref[...], 0.0)
    o_ref[...] = jnp.dot(z1, w2_ref[...],
                         preferred_element_type=jnp.float32) + b2_ref[...]


def _fc(ps, cnt, w1, b1, w2, b2):
    return pl.pallas_call(
        _fc_body,
        out_shape=jax.ShapeDtypeStruct((G, 128), jnp.float32),
    )(ps, cnt, w1, b1, w2, b2)


def _make_agg(D):
    mesh = plsc.VectorSubcoreMesh(core_axis_name="c", subcore_axis_name="s")

    @functools.partial(
        pl.kernel,
        out_type=jax.ShapeDtypeStruct((NP, D), jnp.float32),
        mesh=mesh,
        compiler_params=pltpu.CompilerParams(use_tc_tiling_on_sc=False,
                                             needs_layout_passes=False),
        scratch_types=[
            pltpu.VMEM((2, C + 16), jnp.int32),
            pltpu.VMEM((2, C + 16), jnp.int32),
            pltpu.VMEM((C, D), jnp.float32),
            pltpu.VMEM((C, D), jnp.float32),
            pltpu.VMEM((B, D), jnp.float32),
            pltpu.VMEM((B + 16,), jnp.int32),
            pltpu.SemaphoreType.DMA,
            pltpu.SemaphoreType.DMA,
            pltpu.SemaphoreType.DMA,
        ],
    )
    def agg_kernel(y_hbm, edges_hbm, ro_hbm, out_hbm,
                   ibuf0, ibuf1, gbuf0, gbuf1, acc, robuf,
                   isem, gsem0, gsem1):
        wid = lax.axis_index("s") * 2 + lax.axis_index("c")
        zero16 = jnp.zeros((16,), jnp.float32)
        ibufs = (ibuf0, ibuf1)
        gbufs = (gbuf0, gbuf1)
        gsems = (gsem0, gsem1)

        def subblock(it, carry):
            nb = it * NW + wid

            @pl.when(nb < NBT)
            def _():
                base = nb * B
                pltpu.sync_copy(ro_hbm.at[pl.ds(base, B + 16)], robuf)
                s = robuf[pl.ds(0, 16)][0]
                t = robuf[pl.ds(B, 16)][0]
                a0 = (s // 8) * 8
                nch = (t - a0 + (C - 1)) // C

                def issue_idx(i, b):
                    pltpu.async_copy(
                        edges_hbm.at[:, pl.ds(a0 + i * C, C)],
                        ibufs[b].at[:, pl.ds(0, C)], isem)

                def wait_idx(i, b):
                    pltpu.make_async_copy(
                        edges_hbm.at[:, pl.ds(a0 + i * C, C)],
                        ibufs[b].at[:, pl.ds(0, C)], isem).wait()

                def issue_gather(i, b):
                    pltpu.async_copy(
                        y_hbm.at[ibufs[b].at[0, pl.ds(0, C)]],
                        gbufs[b], gsems[b])

                def wait_gather(b):
                    pltpu.make_async_copy(
                        y_hbm.at[pl.ds(0, C)], gbufs[b], gsems[b]).wait()

                def accumulate(i, b):
                    a = a0 + i * C
                    elo = jnp.maximum(s - a, 0)
                    ehi = jnp.minimum(t - a, C)
                    ib = ibufs[b]
                    gb = gbufs[b]

                    def edge(e, c3):
                        dv = ib[1, pl.ds(e, 16)][0] - base
                        for j in range(D // 16):
                            sl = pl.ds(j * 16, 16)
                            plsc.addupdate(acc.at[dv, sl], gb[e, sl])
                        return c3

                    lax.fori_loop(elo, ehi, edge, 0)

                @pl.when(nch > 0)
                def _():
                    issue_idx(0, 0)

                def zrow(r, c2):
                    for j in range(D // 16):
                        acc[r, pl.ds(j * 16, 16)] = zero16
                    return c2

                lax.fori_loop(0, B, zrow, 0)

                def pair(i2, c2):
                    for b in (0, 1):
                        i = i2 * 2 + b

                        @pl.when(i < nch)
                        def _():
                            wait_idx(i, b)
                            issue_gather(i, b)

                            @pl.when(i > 0)
                            def _():
                                wait_gather(1 - b)
                                accumulate(i - 1, 1 - b)

                            @pl.when(i + 1 < nch)
                            def _():
                                issue_idx(i + 1, 1 - b)
                    return c2

                lax.fori_loop(0, (nch + 1) // 2, pair, 0)

                for b in (0, 1):
                    @pl.when((nch > 0) & ((nch - 1) % 2 == b))
                    def _():
                        wait_gather(b)
                        accumulate(nch - 1, b)

                pltpu.sync_copy(acc.at[pl.ds(0, B)],
                                out_hbm.at[pl.ds(base, B)])

            return carry

        lax.fori_loop(0, SUB_ITERS, subblock, 0)

    return agg_kernel


_agg256 = _make_agg(256)
_agg384 = _make_agg(384)
_agg128 = _make_agg(128)


def kernel(x, edge_index, batch, W1, b1, g1, bt1, W2, b2, g2, bt2,
           W3, b3, g3, bt3, fW1, fb1, fW2, fb2):
    f32 = jnp.float32
    src = edge_index[0]
    dst = edge_index[1]
    dst_s, src_s = lax.sort_key_val(dst, src)
    ro = jnp.searchsorted(
        dst_s, jnp.arange(NP + 16, dtype=jnp.int32), side='left'
    ).astype(jnp.int32)
    src_p = jnp.pad(src_s, (0, EP - E))
    dst_p = jnp.pad(dst_s, (0, EP - E), constant_values=N)
    edges_p = jnp.stack([src_p, dst_p])
    deg = (ro[1:NP + 1] - ro[:NP]).astype(f32) + 1.0
    degb = jnp.broadcast_to(deg[:, None], (NP, 128))
    batch_p = jnp.pad(batch, (0, NP - N), constant_values=G)
    batchb = jnp.broadcast_to(batch_p[:, None], (NP, 128))

    xp = jnp.pad(x, ((0, NP - N), (0, 128 - x.shape[1])))
    W1p = jnp.pad(W1, ((0, 128 - W1.shape[0]), (0, 256 - W1.shape[1])))
    b1p = jnp.pad(b1, (0, 256 - b1.shape[0]))[None, :]
    g1p = jnp.pad(g1, (0, 256 - g1.shape[0]))[None, :]
    bt1p = jnp.pad(bt1, (0, 256 - bt1.shape[0]))[None, :]
    W2p = jnp.pad(W2, ((0, 256 - W2.shape[0]), (0, 384 - W2.shape[1])))
    b2p = jnp.pad(b2, (0, 384 - b2.shape[0]))[None, :]
    g2p = jnp.pad(g2, (0, 384 - g2.shape[0]))[None, :]
    bt2p = jnp.pad(bt2, (0, 384 - bt2.shape[0]))[None, :]
    W3p = jnp.pad(W3, ((0, 384 - W3.shape[0]), (0, 0)))

    y1 = _matmul_scale(xp, W1p, degb, 256)
    agg1 = _agg256(y1, edges_p, ro)
    y2 = _post_mm(156, agg1, y1, degb, b1p, g1p, bt1p, W2p, 384)
    agg2 = _agg384(y2, edges_p, ro)
    y3 = _post_mm(312, agg2, y2, degb, b2p, g2p, bt2p, W3p, 128)
    agg3 = _agg128(y3, edges_p, ro)
    ps, cnt = _post_pool(agg3, y3, degb, b3[None, :], g3[None, :],
                         bt3[None, :], batchb)
    return _fc(ps, cnt, fW1, fb1[None, :], fW2, fb2[None, :])


# speedup vs baseline: 4.3892x; 1.0964x over previous
import functools

import jax
import jax.numpy as jnp
from jax import lax
from jax.experimental import pallas as pl
from jax.experimental.pallas import tpu as pltpu
from jax.experimental.pallas import tpu_sc as plsc

N = 50000
E = 800000
G = 256
NP = 50176
EP = 800256
R = 512
C = 128
B = 64
NBT = NP // B
NW = 32
SUB_ITERS = (NBT + NW - 1) // NW


def _mm_scale_body(x_ref, w_ref, deg_ref, o_ref):
    d = lax.rsqrt(deg_ref[:, :1])
    o_ref[...] = jnp.dot(x_ref[...], w_ref[...],
                         preferred_element_type=jnp.float32) * d


def _matmul_scale(xp, w, degb, dout):
    din = xp.shape[1]
    return pl.pallas_call(
        _mm_scale_body,
        grid=(NP // R,),
        in_specs=[pl.BlockSpec((R, din), lambda i: (i, 0)),
                  pl.BlockSpec((din, dout), lambda i: (0, 0)),
                  pl.BlockSpec((R, 128), lambda i: (i, 0))],
        out_specs=pl.BlockSpec((R, dout), lambda i: (i, 0)),
        out_shape=jax.ShapeDtypeStruct((NP, dout), jnp.float32),
    )(xp, w, degb)


def _post_mm_body(valid, agg_ref, y_ref, deg_ref, b_ref, g_ref, bt_ref,
                  w_ref, o_ref):
    d = lax.rsqrt(deg_ref[:, :1])
    v = jnp.maximum(d * (agg_ref[...] + y_ref[...]) + b_ref[...], 0.0)
    dp = v.shape[1]
    inv = 1.0 / valid
    mu = jnp.sum(v, axis=1, keepdims=True) * inv
    mask = (lax.broadcasted_iota(jnp.int32, (1, dp), 1) < valid
            ).astype(jnp.float32)
    c = (v - mu) * mask
    var = jnp.sum(c * c, axis=1, keepdims=True) * inv
    h = c * lax.rsqrt(var + 1e-5) * g_ref[...] + bt_ref[...]
    o_ref[...] = jnp.dot(h, w_ref[...],
                         preferred_element_type=jnp.float32) * d


def _post_mm(valid, agg, y, degb, bp, gp, btp, w, dout):
    dp = y.shape[1]
    return pl.pallas_call(
        functools.partial(_post_mm_body, float(valid)),
        grid=(NP // R,),
        in_specs=[pl.BlockSpec((R, dp), lambda i: (i, 0)),
                  pl.BlockSpec((R, dp), lambda i: (i, 0)),
                  pl.BlockSpec((R, 128), lambda i: (i, 0)),
                  pl.BlockSpec((1, dp), lambda i: (0, 0)),
                  pl.BlockSpec((1, dp), lambda i: (0, 0)),
                  pl.BlockSpec((1, dp), lambda i: (0, 0)),
                  pl.BlockSpec((dp, dout), lambda i: (0, 0))],
        out_specs=pl.BlockSpec((R, dout), lambda i: (i, 0)),
        out_shape=jax.ShapeDtypeStruct((NP, dout), jnp.float32),
    )(agg, y, degb, bp, gp, btp, w)


def _post_pool_body(agg_ref, y_ref, deg_ref, b_ref, g_ref, bt_ref,
                    batch_ref, ps_ref, cnt_ref):
    i = pl.program_id(0)
    d = lax.rsqrt(deg_ref[:, :1])
    v = jnp.maximum(d * (agg_ref[...] + y_ref[...]) + b_ref[...], 0.0)
    mu = jnp.mean(v, axis=1, keepdims=True)
    c = v - mu
    var = jnp.mean(c * c, axis=1, keepdims=True)
    h = c * lax.rsqrt(var + 1e-5) * g_ref[...] + bt_ref[...]
    onehot = (batch_ref[:, :1] == lax.broadcasted_iota(jnp.int32, (1, G), 1)
              ).astype(jnp.float32)
    dn = (((0,), (0,)), ((), ()))
    ps = lax.dot_general(onehot, h, dn, preferred_element_type=jnp.float32)
    cn = lax.dot_general(onehot, jnp.ones((R, 128), jnp.float32), dn,
                         preferred_element_type=jnp.float32)

    @pl.when(i == 0)
    def _():
        ps_ref[...] = ps
        cnt_ref[...] = cn

    @pl.when(i != 0)
    def _():
        ps_ref[...] += ps
        cnt_ref[...] += cn


def _post_pool(agg, y, degb, bp, gp, btp, batchb):
    return pl.pallas_call(
        _post_pool_body,
        grid=(NP // R,),
        in_specs=[pl.BlockSpec((R, 128), lambda i: (i, 0)),
                  pl.BlockSpec((R, 128), lambda i: (i, 0)),
                  pl.BlockSpec((R, 128), lambda i: (i, 0)),
                  pl.BlockSpec((1, 128), lambda i: (0, 0)),
                  pl.BlockSpec((1, 128), lambda i: (0, 0)),
                  pl.BlockSpec((1, 128), lambda i: (0, 0)),
                  pl.BlockSpec((R, 128), lambda i: (i, 0))],
        out_specs=[pl.BlockSpec((G, 128), lambda i: (0, 0)),
                   pl.BlockSpec((G, 128), lambda i: (0, 0))],
        out_shape=[jax.ShapeDtypeStruct((G, 128), jnp.float32),
                   jax.ShapeDtypeStruct((G, 128), jnp.float32)],
    )(agg, y, degb, bp, gp, btp, batchb)


def _fc_body(ps_ref, cnt_ref, w1_ref, b1_ref, w2_ref, b2_ref, o_ref):
    pooled = ps_ref[...] / jnp.maximum(cnt_ref[...], 1.0)
    z1 = jnp.maximum(
        jnp.dot(pooled, w1_ref[...], preferred_element_type=jnp.float32)
        + b1_ref[...], 0.0)
    o_ref[...] = jnp.dot(z1, w2_ref[...],
                         preferred_element_type=jnp.float32) + b2_ref[...]


def _fc(ps, cnt, w1, b1, w2, b2):
    return pl.pallas_call(
        _fc_body,
        out_shape=jax.ShapeDtypeStruct((G, 128), jnp.float32),
    )(ps, cnt, w1, b1, w2, b2)


def _make_agg(D):
    mesh = plsc.VectorSubcoreMesh(core_axis_name="c", subcore_axis_name="s")

    @functools.partial(
        pl.kernel,
        out_type=jax.ShapeDtypeStruct((NP, D), jnp.float32),
        mesh=mesh,
        compiler_params=pltpu.CompilerParams(use_tc_tiling_on_sc=False,
                                             needs_layout_passes=False),
        scratch_types=[
            pltpu.VMEM((2, C + 16), jnp.int32),
            pltpu.VMEM((2, C + 16), jnp.int32),
            pltpu.VMEM((C, D), jnp.float32),
            pltpu.VMEM((C, D), jnp.float32),
            pltpu.VMEM((B, D), jnp.float32),
            pltpu.VMEM((B + 16,), jnp.int32),
            pltpu.SemaphoreType.DMA,
            pltpu.SemaphoreType.DMA,
            pltpu.SemaphoreType.DMA,
        ],
    )
    def agg_kernel(y_hbm, edges_hbm, ro_hbm, out_hbm,
                   ibuf0, ibuf1, gbuf0, gbuf1, acc, robuf,
                   isem, gsem0, gsem1):
        wid = lax.axis_index("s") * 2 + lax.axis_index("c")
        zero16 = jnp.zeros((16,), jnp.float32)
        ibufs = (ibuf0, ibuf1)
        gbufs = (gbuf0, gbuf1)
        gsems = (gsem0, gsem1)

        def subblock(it, carry):
            nb = it * NW + wid

            @pl.when(nb < NBT)
            def _():
                base = nb * B
                pltpu.sync_copy(ro_hbm.at[pl.ds(base, B + 16)], robuf)
                s = robuf[pl.ds(0, 16)][0]
                t = robuf[pl.ds(B, 16)][0]
                a0 = (s // 8) * 8
                nch = (t - a0 + (C - 1)) // C

                def issue_idx(i, b):
                    pltpu.async_copy(
                        edges_hbm.at[:, pl.ds(a0 + i * C, C)],
                        ibufs[b].at[:, pl.ds(0, C)], isem)

                def wait_idx(i, b):
                    pltpu.make_async_copy(
                        edges_hbm.at[:, pl.ds(a0 + i * C, C)],
                        ibufs[b].at[:, pl.ds(0, C)], isem).wait()

                def issue_gather(i, b):
                    pltpu.async_copy(
                        y_hbm.at[ibufs[b].at[0, pl.ds(0, C)]],
                        gbufs[b], gsems[b])

                def wait_gather(b):
                    pltpu.make_async_copy(
                        y_hbm.at[pl.ds(0, C)], gbufs[b], gsems[b]).wait()

                def accumulate(i, b):
                    a = a0 + i * C
                    elo = jnp.maximum(s - a, 0)
                    ehi = jnp.minimum(t - a, C)
                    ib = ibufs[b]
                    gb = gbufs[b]

                    def edge(e, c3):
                        dv = ib[1, pl.ds(e, 16)][0] - base
                        for j in range(D // 16):
                            sl = pl.ds(j * 16, 16)
                            plsc.addupdate(acc.at[dv, sl], gb[e, sl])
                        return c3

                    lax.fori_loop(elo, ehi, edge, 0)

                @pl.when(nch > 0)
                def _():
                    issue_idx(0, 0)

                def zrow(r, c2):
                    for j in range(D // 16):
                        acc[r, pl.ds(j * 16, 16)] = zero16
                    return c2

                lax.fori_loop(0, B, zrow, 0)

                def pair(i2, c2):
                    for b in (0, 1):
                        i = i2 * 2 + b

                        @pl.when(i < nch)
                        def _():
                            wait_idx(i, b)
                            issue_gather(i, b)

                            @pl.when(i > 0)
                            def _():
                                wait_gather(1 - b)
                                accumulate(i - 1, 1 - b)

                            @pl.when(i + 1 < nch)
                            def _():
                                issue_idx(i + 1, 1 - b)
                    return c2

                lax.fori_loop(0, (nch + 1) // 2, pair, 0)

                for b in (0, 1):
                    @pl.when((nch > 0) & ((nch - 1) % 2 == b))
                    def _():
                        wait_gather(b)
                        accumulate(nch - 1, b)

                pltpu.sync_copy(acc.at[pl.ds(0, B)],
                                out_hbm.at[pl.ds(base, B)])

            return carry

        lax.fori_loop(0, SUB_ITERS, subblock, 0)

    return agg_kernel


_agg160 = _make_agg(160)
_agg320 = _make_agg(320)
_agg128 = _make_agg(128)


def kernel(x, edge_index, batch, W1, b1, g1, bt1, W2, b2, g2, bt2,
           W3, b3, g3, bt3, fW1, fb1, fW2, fb2):
    f32 = jnp.float32
    src = edge_index[0]
    dst = edge_index[1]
    dst_s, src_s = lax.sort_key_val(dst, src)
    ro = jnp.searchsorted(
        dst_s, jnp.arange(NP + 16, dtype=jnp.int32), side='left'
    ).astype(jnp.int32)
    src_p = jnp.pad(src_s, (0, EP - E))
    dst_p = jnp.pad(dst_s, (0, EP - E), constant_values=N)
    edges_p = jnp.stack([src_p, dst_p])
    deg = (ro[1:NP + 1] - ro[:NP]).astype(f32) + 1.0
    degb = jnp.broadcast_to(deg[:, None], (NP, 128))
    batch_p = jnp.pad(batch, (0, NP - N), constant_values=G)
    batchb = jnp.broadcast_to(batch_p[:, None], (NP, 128))

    xp = jnp.pad(x, ((0, NP - N), (0, 128 - x.shape[1])))
    W1p = jnp.pad(W1, ((0, 128 - W1.shape[0]), (0, 160 - W1.shape[1])))
    b1p = jnp.pad(b1, (0, 160 - b1.shape[0]))[None, :]
    g1p = jnp.pad(g1, (0, 160 - g1.shape[0]))[None, :]
    bt1p = jnp.pad(bt1, (0, 160 - bt1.shape[0]))[None, :]
    W2p = jnp.pad(W2, ((0, 160 - W2.shape[0]), (0, 320 - W2.shape[1])))
    b2p = jnp.pad(b2, (0, 320 - b2.shape[0]))[None, :]
    g2p = jnp.pad(g2, (0, 320 - g2.shape[0]))[None, :]
    bt2p = jnp.pad(bt2, (0, 320 - bt2.shape[0]))[None, :]
    W3p = jnp.pad(W3, ((0, 320 - W3.shape[0]), (0, 0)))

    y1 = _matmul_scale(xp, W1p, degb, 160)
    agg1 = _agg160(y1, edges_p, ro)
    y2 = _post_mm(156, agg1, y1, degb, b1p, g1p, bt1p, W2p, 320)
    agg2 = _agg320(y2, edges_p, ro)
    y3 = _post_mm(312, agg2, y2, degb, b2p, g2p, bt2p, W3p, 128)
    agg3 = _agg128(y3, edges_p, ro)
    ps, cnt = _post_pool(agg3, y3, degb, b3[None, :], g3[None, :],
                         bt3[None, :], batchb)
    return _fc(ps, cnt, fW1, fb1[None, :], fW2, fb2[None, :])


# speedup vs baseline: 6.8821x; 1.5680x over previous
import functools

import jax
import jax.numpy as jnp
from jax import lax
from jax.experimental import pallas as pl
from jax.experimental.pallas import tpu as pltpu
from jax.experimental.pallas import tpu_sc as plsc

N = 50000
E = 800000
G = 256
NP = 50176
EP = 800256
R = 512
C = 128
B = 64
NBT = NP // B
NW = 32
SUB_ITERS = (NBT + NW - 1) // NW


def _mm_scale_body(x_ref, w_ref, deg_ref, o_ref):
    d = lax.rsqrt(deg_ref[:, :1])
    o_ref[...] = jnp.dot(x_ref[...], w_ref[...],
                         preferred_element_type=jnp.float32) * d


def _matmul_scale(xp, w, degb, dout):
    din = xp.shape[1]
    return pl.pallas_call(
        _mm_scale_body,
        grid=(NP // R,),
        in_specs=[pl.BlockSpec((R, din), lambda i: (i, 0)),
                  pl.BlockSpec((din, dout), lambda i: (0, 0)),
                  pl.BlockSpec((R, 128), lambda i: (i, 0))],
        out_specs=pl.BlockSpec((R, dout), lambda i: (i, 0)),
        out_shape=jax.ShapeDtypeStruct((NP, dout), jnp.float32),
    )(xp, w, degb)


def _post_mm_body(valid, agg_ref, y_ref, deg_ref, b_ref, g_ref, bt_ref,
                  w_ref, o_ref):
    d = lax.rsqrt(deg_ref[:, :1])
    v = jnp.maximum(d * (agg_ref[...] + y_ref[...]) + b_ref[...], 0.0)
    dp = v.shape[1]
    inv = 1.0 / valid
    mu = jnp.sum(v, axis=1, keepdims=True) * inv
    mask = (lax.broadcasted_iota(jnp.int32, (1, dp), 1) < valid
            ).astype(jnp.float32)
    c = (v - mu) * mask
    var = jnp.sum(c * c, axis=1, keepdims=True) * inv
    h = c * lax.rsqrt(var + 1e-5) * g_ref[...] + bt_ref[...]
    o_ref[...] = jnp.dot(h, w_ref[...],
                         preferred_element_type=jnp.float32) * d


def _post_mm(valid, agg, y, degb, bp, gp, btp, w, dout):
    dp = y.shape[1]
    return pl.pallas_call(
        functools.partial(_post_mm_body, float(valid)),
        grid=(NP // R,),
        in_specs=[pl.BlockSpec((R, dp), lambda i: (i, 0)),
                  pl.BlockSpec((R, dp), lambda i: (i, 0)),
                  pl.BlockSpec((R, 128), lambda i: (i, 0)),
                  pl.BlockSpec((1, dp), lambda i: (0, 0)),
                  pl.BlockSpec((1, dp), lambda i: (0, 0)),
                  pl.BlockSpec((1, dp), lambda i: (0, 0)),
                  pl.BlockSpec((dp, dout), lambda i: (0, 0))],
        out_specs=pl.BlockSpec((R, dout), lambda i: (i, 0)),
        out_shape=jax.ShapeDtypeStruct((NP, dout), jnp.float32),
    )(agg, y, degb, bp, gp, btp, w)


def _post_pool_body(agg_ref, y_ref, deg_ref, b_ref, g_ref, bt_ref,
                    batch_ref, ps_ref, cnt_ref):
    i = pl.program_id(0)
    d = lax.rsqrt(deg_ref[:, :1])
    v = jnp.maximum(d * (agg_ref[...] + y_ref[...]) + b_ref[...], 0.0)
    mu = jnp.mean(v, axis=1, keepdims=True)
    c = v - mu
    var = jnp.mean(c * c, axis=1, keepdims=True)
    h = c * lax.rsqrt(var + 1e-5) * g_ref[...] + bt_ref[...]
    onehot = (batch_ref[:, :1] == lax.broadcasted_iota(jnp.int32, (1, G), 1)
              ).astype(jnp.float32)
    dn = (((0,), (0,)), ((), ()))
    ps = lax.dot_general(onehot, h, dn, preferred_element_type=jnp.float32)
    cn = lax.dot_general(onehot, jnp.ones((R, 128), jnp.float32), dn,
                         preferred_element_type=jnp.float32)

    @pl.when(i == 0)
    def _():
        ps_ref[...] = ps
        cnt_ref[...] = cn

    @pl.when(i != 0)
    def _():
        ps_ref[...] += ps
        cnt_ref[...] += cn


def _post_pool(agg, y, degb, bp, gp, btp, batchb):
    return pl.pallas_call(
        _post_pool_body,
        grid=(NP // R,),
        in_specs=[pl.BlockSpec((R, 128), lambda i: (i, 0)),
                  pl.BlockSpec((R, 128), lambda i: (i, 0)),
                  pl.BlockSpec((R, 128), lambda i: (i, 0)),
                  pl.BlockSpec((1, 128), lambda i: (0, 0)),
                  pl.BlockSpec((1, 128), lambda i: (0, 0)),
                  pl.BlockSpec((1, 128), lambda i: (0, 0)),
                  pl.BlockSpec((R, 128), lambda i: (i, 0))],
        out_specs=[pl.BlockSpec((G, 128), lambda i: (0, 0)),
                   pl.BlockSpec((G, 128), lambda i: (0, 0))],
        out_shape=[jax.ShapeDtypeStruct((G, 128), jnp.float32),
                   jax.ShapeDtypeStruct((G, 128), jnp.float32)],
    )(agg, y, degb, bp, gp, btp, batchb)


def _fc_body(ps_ref, cnt_ref, w1_ref, b1_ref, w2_ref, b2_ref, o_ref):
    pooled = ps_ref[...] / jnp.maximum(cnt_ref[...], 1.0)
    z1 = jnp.maximum(
        jnp.dot(pooled, w1_ref[...], preferred_element_type=jnp.float32)
        + b1_ref[...], 0.0)
    o_ref[...] = jnp.dot(z1, w2_ref[...],
                         preferred_element_type=jnp.float32) + b2_ref[...]


def _fc(ps, cnt, w1, b1, w2, b2):
    return pl.pallas_call(
        _fc_body,
        out_shape=jax.ShapeDtypeStruct((G, 128), jnp.float32),
    )(ps, cnt, w1, b1, w2, b2)


def _make_agg(D):
    mesh = plsc.VectorSubcoreMesh(core_axis_name="c", subcore_axis_name="s")

    @functools.partial(
        pl.kernel,
        out_type=jax.ShapeDtypeStruct((NP, D), jnp.float32),
        mesh=mesh,
        compiler_params=pltpu.CompilerParams(use_tc_tiling_on_sc=False,
                                             needs_layout_passes=False),
        scratch_types=[
            pltpu.VMEM((2, C + 16), jnp.int32),
            pltpu.VMEM((2, C + 16), jnp.int32),
            pltpu.VMEM((C, D), jnp.float32),
            pltpu.VMEM((C, D), jnp.float32),
            pltpu.VMEM((B, D), jnp.float32),
            pltpu.VMEM((B + 16,), jnp.int32),
            pltpu.SemaphoreType.DMA,
            pltpu.SemaphoreType.DMA,
            pltpu.SemaphoreType.DMA,
        ],
    )
    def agg_kernel(y_hbm, edges_hbm, ro_hbm, out_hbm,
                   ibuf0, ibuf1, gbuf0, gbuf1, acc, robuf,
                   isem, gsem0, gsem1):
        wid = lax.axis_index("s") * 2 + lax.axis_index("c")
        zero16 = jnp.zeros((16,), jnp.float32)
        ibufs = (ibuf0, ibuf1)
        gbufs = (gbuf0, gbuf1)
        gsems = (gsem0, gsem1)

        def subblock(it, carry):
            nb = it * NW + wid

            @pl.when(nb < NBT)
            def _():
                base = nb * B
                pltpu.sync_copy(ro_hbm.at[pl.ds(base, B + 16)], robuf)
                s = robuf[pl.ds(0, 16)][0]
                t = robuf[pl.ds(B, 16)][0]
                a0 = (s // 8) * 8
                nch = (t - a0 + (C - 1)) // C

                def issue_idx(i, b):
                    pltpu.async_copy(
                        edges_hbm.at[:, pl.ds(a0 + i * C, C)],
                        ibufs[b].at[:, pl.ds(0, C)], isem)

                def wait_idx(i, b):
                    pltpu.make_async_copy(
                        edges_hbm.at[:, pl.ds(a0 + i * C, C)],
                        ibufs[b].at[:, pl.ds(0, C)], isem).wait()

                def issue_gather(i, b):
                    pltpu.async_copy(
                        y_hbm.at[ibufs[b].at[0, pl.ds(0, C)]],
                        gbufs[b], gsems[b])

                def wait_gather(b):
                    pltpu.make_async_copy(
                        y_hbm.at[pl.ds(0, C)], gbufs[b], gsems[b]).wait()

                def accumulate(i, b):
                    a = a0 + i * C
                    elo = jnp.maximum(s - a, 0)
                    ehi = jnp.minimum(t - a, C)
                    ib = ibufs[b]
                    gb = gbufs[b]

                    @pl.when(ehi > elo)
                    def _():
                        dfirst = ib[1, pl.ds(elo, 16)][0] - base
                        dlast = ib[1, pl.ds(ehi - 1, 16)][0] - base

                        def node(dn, c4):
                            rlo = robuf[pl.ds(dn, 16)][0]
                            rhi = robuf[pl.ds(dn + 1, 16)][0]
                            lo_e = jnp.maximum(rlo - a, 0)
                            hi_e = jnp.minimum(rhi - a, C)

                            def edge(e, regs):
                                return tuple(
                                    regs[j] + gb[e, pl.ds(j * 16, 16)]
                                    for j in range(D // 16))

                            zregs = tuple(jnp.zeros((16,), jnp.float32)
                                          for _ in range(D // 16))
                            regs = lax.fori_loop(lo_e, hi_e, edge, zregs)
                            for j in range(D // 16):
                                plsc.addupdate(
                                    acc.at[dn, pl.ds(j * 16, 16)], regs[j])
                            return c4

                        lax.fori_loop(dfirst, dlast + 1, node, 0)

                @pl.when(nch > 0)
                def _():
                    issue_idx(0, 0)

                def zrow(r, c2):
                    for j in range(D // 16):
                        acc[r, pl.ds(j * 16, 16)] = zero16
                    return c2

                lax.fori_loop(0, B, zrow, 0)

                def pair(i2, c2):
                    for b in (0, 1):
                        i = i2 * 2 + b

                        @pl.when(i < nch)
                        def _():
                            wait_idx(i, b)
                            issue_gather(i, b)

                            @pl.when(i > 0)
                            def _():
                                wait_gather(1 - b)
                                accumulate(i - 1, 1 - b)

                            @pl.when(i + 1 < nch)
                            def _():
                                issue_idx(i + 1, 1 - b)
                    return c2

                lax.fori_loop(0, (nch + 1) // 2, pair, 0)

                for b in (0, 1):
                    @pl.when((nch > 0) & ((nch - 1) % 2 == b))
                    def _():
                        wait_gather(b)
                        accumulate(nch - 1, b)

                pltpu.sync_copy(acc.at[pl.ds(0, B)],
                                out_hbm.at[pl.ds(base, B)])

            return carry

        lax.fori_loop(0, SUB_ITERS, subblock, 0)

    return agg_kernel


_agg160 = _make_agg(160)
_agg320 = _make_agg(320)
_agg128 = _make_agg(128)


def kernel(x, edge_index, batch, W1, b1, g1, bt1, W2, b2, g2, bt2,
           W3, b3, g3, bt3, fW1, fb1, fW2, fb2):
    f32 = jnp.float32
    src = edge_index[0]
    dst = edge_index[1]
    dst_s, src_s = lax.sort_key_val(dst, src)
    ro = jnp.searchsorted(
        dst_s, jnp.arange(NP + 16, dtype=jnp.int32), side='left'
    ).astype(jnp.int32)
    src_p = jnp.pad(src_s, (0, EP - E))
    dst_p = jnp.pad(dst_s, (0, EP - E), constant_values=N)
    edges_p = jnp.stack([src_p, dst_p])
    deg = (ro[1:NP + 1] - ro[:NP]).astype(f32) + 1.0
    degb = jnp.broadcast_to(deg[:, None], (NP, 128))
    batch_p = jnp.pad(batch, (0, NP - N), constant_values=G)
    batchb = jnp.broadcast_to(batch_p[:, None], (NP, 128))

    xp = jnp.pad(x, ((0, NP - N), (0, 128 - x.shape[1])))
    W1p = jnp.pad(W1, ((0, 128 - W1.shape[0]), (0, 160 - W1.shape[1])))
    b1p = jnp.pad(b1, (0, 160 - b1.shape[0]))[None, :]
    g1p = jnp.pad(g1, (0, 160 - g1.shape[0]))[None, :]
    bt1p = jnp.pad(bt1, (0, 160 - bt1.shape[0]))[None, :]
    W2p = jnp.pad(W2, ((0, 160 - W2.shape[0]), (0, 320 - W2.shape[1])))
    b2p = jnp.pad(b2, (0, 320 - b2.shape[0]))[None, :]
    g2p = jnp.pad(g2, (0, 320 - g2.shape[0]))[None, :]
    bt2p = jnp.pad(bt2, (0, 320 - bt2.shape[0]))[None, :]
    W3p = jnp.pad(W3, ((0, 320 - W3.shape[0]), (0, 0)))

    y1 = _matmul_scale(xp, W1p, degb, 160)
    agg1 = _agg160(y1, edges_p, ro)
    y2 = _post_mm(156, agg1, y1, degb, b1p, g1p, bt1p, W2p, 320)
    agg2 = _agg320(y2, edges_p, ro)
    y3 = _post_mm(312, agg2, y2, degb, b2p, g2p, bt2p, W3p, 128)
    agg3 = _agg128(y3, edges_p, ro)
    ps, cnt = _post_pool(agg3, y3, degb, b3[None, :], g3[None, :],
                         bt3[None, :], batchb)
    return _fc(ps, cnt, fW1, fb1[None, :], fW2, fb2[None, :])
